# Initial kernel scaffold; baseline (speedup 1.0000x reference)
import jax
import jax.numpy as jnp
from jax.experimental import pallas as pl


def kernel(tokens, table):
    raise NotImplementedError("write your pallas kernel here")


# speedup vs baseline: 8.3570x; 8.3570x over previous
import functools

import jax
import jax.numpy as jnp
from jax import lax
from jax.experimental import pallas as pl
from jax.experimental.pallas import tpu as pltpu
from jax.experimental.pallas import tpu_sc as plsc

N_MODULES = 500
AUX_TOKENS = 30
VOCAB = N_MODULES + AUX_TOKENS
EMBED_DIM = 64
BATCH = 4096
SEQ_LEN = 500

B = BATCH * SEQ_LEN
NUM_WORKERS = 32
B_PER_W = B // NUM_WORKERS
CHUNK = 512
GATHER = 128
N_ITERS = B_PER_W // CHUNK
N_GATHER = CHUNK // GATHER


def _sc_gather(tokens_flat, table):
    mesh = plsc.VectorSubcoreMesh(core_axis_name="c", subcore_axis_name="s")

    @functools.partial(
        pl.kernel,
        mesh=mesh,
        out_type=jax.ShapeDtypeStruct((B, EMBED_DIM), jnp.float32),
        scratch_types=[
            pltpu.VMEM((CHUNK,), jnp.int32),
            pltpu.VMEM((CHUNK, EMBED_DIM), jnp.float32),
            pltpu.SemaphoreType.DMA,
        ],
        compiler_params=pltpu.CompilerParams(use_tc_tiling_on_sc=False),
    )
    def k(tok_hbm, table_hbm, out_hbm, idx_v, rows_v, sem):
        wid = lax.axis_index("s") * 2 + lax.axis_index("c")
        w_base = wid * B_PER_W

        def body(g, carry):
            base = w_base + g * CHUNK
            pltpu.sync_copy(tok_hbm.at[pl.ds(base, CHUNK)], idx_v)
            for j in range(N_GATHER):
                pltpu.async_copy(
                    table_hbm.at[idx_v.at[pl.ds(j * GATHER, GATHER)]],
                    rows_v.at[pl.ds(j * GATHER, GATHER)],
                    sem,
                ).wait()
            pltpu.sync_copy(rows_v, out_hbm.at[pl.ds(base, CHUNK)])
            return carry

        lax.fori_loop(0, N_ITERS, body, 0)

    return k(tokens_flat, table)


def kernel(tokens, table):
    out = _sc_gather(tokens.reshape(B), table)
    return out.reshape(BATCH, SEQ_LEN, EMBED_DIM)


# speedup vs baseline: 8.4015x; 1.0053x over previous
import functools

import jax
import jax.numpy as jnp
from jax import lax
from jax.experimental import pallas as pl
from jax.experimental.pallas import tpu as pltpu
from jax.experimental.pallas import tpu_sc as plsc

N_MODULES = 500
AUX_TOKENS = 30
VOCAB = N_MODULES + AUX_TOKENS
EMBED_DIM = 64
BATCH = 4096
SEQ_LEN = 500

B = BATCH * SEQ_LEN
NUM_WORKERS = 32
B_PER_W = B // NUM_WORKERS
CHUNK = 640
GATHER = 128
N_GATHER = CHUNK // GATHER
NBUF = 2
N_OUTER = B_PER_W // (CHUNK * NBUF)


def _sc_gather(tokens_flat, table):
    mesh = plsc.VectorSubcoreMesh(core_axis_name="c", subcore_axis_name="s")

    @functools.partial(
        pl.kernel,
        mesh=mesh,
        out_type=jax.ShapeDtypeStruct((B, EMBED_DIM), jnp.float32),
        scratch_types=[
            [pltpu.VMEM((CHUNK,), jnp.int32)] * NBUF,
            [pltpu.VMEM((CHUNK, EMBED_DIM), jnp.float32)] * NBUF,
            [pltpu.SemaphoreType.DMA] * NBUF,
            [pltpu.SemaphoreType.DMA] * NBUF,
        ],
        compiler_params=pltpu.CompilerParams(use_tc_tiling_on_sc=False),
    )
    def k(tok_hbm, table_hbm, out_hbm, idx_v, rows_v, sem_g, sem_o):
        wid = lax.axis_index("s") * 2 + lax.axis_index("c")
        w_base = wid * B_PER_W

        def body(t, carry):
            for b in range(NBUF):
                base = w_base + (t * NBUF + b) * CHUNK

                @pl.when(t > 0)
                def _drain_prev_write(b=b):
                    pltpu.make_async_copy(
                        rows_v[b], out_hbm.at[pl.ds(0, CHUNK)], sem_o[b]
                    ).wait()

                pltpu.sync_copy(tok_hbm.at[pl.ds(base, CHUNK)], idx_v[b])
                for j in range(N_GATHER):
                    pltpu.async_copy(
                        table_hbm.at[idx_v[b].at[pl.ds(j * GATHER, GATHER)]],
                        rows_v[b].at[pl.ds(j * GATHER, GATHER)],
                        sem_g[b],
                    )
            for b in range(NBUF):
                base = w_base + (t * NBUF + b) * CHUNK
                for j in range(N_GATHER):
                    pltpu.make_async_copy(
                        table_hbm.at[idx_v[b].at[pl.ds(j * GATHER, GATHER)]],
                        rows_v[b].at[pl.ds(j * GATHER, GATHER)],
                        sem_g[b],
                    ).wait()
                pltpu.async_copy(rows_v[b], out_hbm.at[pl.ds(base, CHUNK)], sem_o[b])
            return carry

        lax.fori_loop(0, N_OUTER, body, 0)
        for b in range(NBUF):
            pltpu.make_async_copy(
                rows_v[b], out_hbm.at[pl.ds(0, CHUNK)], sem_o[b]
            ).wait()

    return k(tokens_flat, table)


def kernel(tokens, table):
    out = _sc_gather(tokens.reshape(B), table)
    return out.reshape(BATCH, SEQ_LEN, EMBED_DIM)


# speedup vs baseline: 8.4065x; 1.0006x over previous
import functools

import jax
import jax.numpy as jnp
from jax import lax
from jax.experimental import pallas as pl
from jax.experimental.pallas import tpu as pltpu
from jax.experimental.pallas import tpu_sc as plsc

N_MODULES = 500
AUX_TOKENS = 30
VOCAB = N_MODULES + AUX_TOKENS
EMBED_DIM = 64
BATCH = 4096
SEQ_LEN = 500

B = BATCH * SEQ_LEN
NUM_WORKERS = 32
B_PER_W = B // NUM_WORKERS
CHUNK = 400
NBUF = 2
N_OUTER = B_PER_W // (CHUNK * NBUF)


def _sc_gather(tokens_flat, table):
    mesh = plsc.VectorSubcoreMesh(core_axis_name="c", subcore_axis_name="s")

    @functools.partial(
        pl.kernel,
        mesh=mesh,
        out_type=jax.ShapeDtypeStruct((B, EMBED_DIM), jnp.float32),
        scratch_types=[
            pltpu.VMEM((B_PER_W,), jnp.int32),
            [pltpu.VMEM((CHUNK, EMBED_DIM), jnp.float32)] * NBUF,
            [pltpu.SemaphoreType.DMA] * NBUF,
            [pltpu.SemaphoreType.DMA] * NBUF,
        ],
        compiler_params=pltpu.CompilerParams(use_tc_tiling_on_sc=False),
    )
    def k(tok_hbm, table_hbm, out_hbm, idx_v, rows_v, sem_g, sem_o):
        wid = lax.axis_index("s") * 2 + lax.axis_index("c")
        w_base = wid * B_PER_W

        pltpu.sync_copy(tok_hbm.at[pl.ds(w_base, B_PER_W)], idx_v)

        def body(t, carry):
            for b in range(NBUF):
                off = (t * NBUF + b) * CHUNK

                @pl.when(t > 0)
                def _drain_prev_write(b=b):
                    pltpu.make_async_copy(
                        rows_v[b], out_hbm.at[pl.ds(0, CHUNK)], sem_o[b]
                    ).wait()

                pltpu.async_copy(
                    table_hbm.at[idx_v.at[pl.ds(off, CHUNK)]],
                    rows_v[b],
                    sem_g[b],
                )
            for b in range(NBUF):
                off = (t * NBUF + b) * CHUNK
                pltpu.make_async_copy(
                    table_hbm.at[idx_v.at[pl.ds(off, CHUNK)]],
                    rows_v[b],
                    sem_g[b],
                ).wait()
                pltpu.async_copy(
                    rows_v[b], out_hbm.at[pl.ds(w_base + off, CHUNK)], sem_o[b]
                )
            return carry

        lax.fori_loop(0, N_OUTER, body, 0)
        for b in range(NBUF):
            pltpu.make_async_copy(
                rows_v[b], out_hbm.at[pl.ds(0, CHUNK)], sem_o[b]
            ).wait()

    return k(tokens_flat, table)


def kernel(tokens, table):
    out = _sc_gather(tokens.reshape(B), table)
    return out.reshape(BATCH, SEQ_LEN, EMBED_DIM)


# speedup vs baseline: 8.6835x; 1.0329x over previous
import functools

import jax
import jax.numpy as jnp
from jax import lax
from jax.experimental import pallas as pl
from jax.experimental.pallas import tpu as pltpu
from jax.experimental.pallas import tpu_sc as plsc

EMBED_DIM = 64
BATCH = 4096
SEQ_LEN = 500

NUM_WORKERS = 32
B_PER_W = BATCH // NUM_WORKERS
NBUF = 2


def _sc_gather(tokens, table):
    mesh = plsc.VectorSubcoreMesh(core_axis_name="c", subcore_axis_name="s")

    @functools.partial(
        pl.kernel,
        mesh=mesh,
        out_type=jax.ShapeDtypeStruct((BATCH, SEQ_LEN, EMBED_DIM), jnp.float32),
        scratch_types=[
            [pltpu.VMEM((SEQ_LEN,), jnp.int32)] * NBUF,
            [pltpu.VMEM((SEQ_LEN, EMBED_DIM), jnp.float32)] * NBUF,
            [pltpu.SemaphoreType.DMA] * NBUF,
            [pltpu.SemaphoreType.DMA] * NBUF,
        ],
        compiler_params=pltpu.CompilerParams(use_tc_tiling_on_sc=False),
    )
    def k(tok_hbm, table_hbm, out_hbm, idx_v, rows_v, sem_g, sem_o):
        wid = lax.axis_index("s") * 2 + lax.axis_index("c")
        w_base = wid * B_PER_W

        def body(t, carry):
            for b in range(NBUF):
                bat = w_base + t * NBUF + b

                @pl.when(t > 0)
                def _drain_prev_write(b=b, bat=bat):
                    pltpu.make_async_copy(
                        rows_v[b], out_hbm.at[bat], sem_o[b]
                    ).wait()

                pltpu.sync_copy(tok_hbm.at[bat], idx_v[b])
                pltpu.async_copy(
                    table_hbm.at[idx_v[b]], rows_v[b], sem_g[b]
                )
            for b in range(NBUF):
                bat = w_base + t * NBUF + b
                pltpu.make_async_copy(
                    table_hbm.at[idx_v[b]], rows_v[b], sem_g[b]
                ).wait()
                pltpu.async_copy(rows_v[b], out_hbm.at[bat], sem_o[b])
            return carry

        lax.fori_loop(0, B_PER_W // NBUF, body, 0)
        for b in range(NBUF):
            pltpu.make_async_copy(
                rows_v[b], out_hbm.at[w_base], sem_o[b]
            ).wait()

    return k(tokens, table)


def kernel(tokens, table):
    return _sc_gather(tokens, table)
